# baseline (speedup 1.0000x reference)
import jax
import jax.numpy as jnp
from jax.experimental import pallas as pl
from jax.experimental.pallas import tpu as pltpu

_E = 8
_B = 1024
_C, _H, _W = 3, 32, 32
_D = _C * _H * _W
_HID = 512
_LAT = 128
_CLS = 100
_TEMP = 2.0
_BT = 256
_NB = _B // _BT

_W_SHAPES = [(_D, _HID), (_HID, _HID), (_HID, _LAT),
             (_LAT, _HID), (_HID, _HID), (_HID, _D),
             (_LAT, _HID), (_HID, _HID), (_HID, _CLS)]


def _expert_body(flat_hbm, We1_hbm, We2_hbm, We3_hbm, Wd1_hbm, Wd2_hbm, Wd3_hbm,
                 Wx1_hbm, Wx2_hbm, Wx3_hbm,
                 be1_ref, be2_ref, be3_ref, bd1_ref, bd2_ref, bd3_ref,
                 bx1_ref, bx2_ref, bx3_ref,
                 recon_ref, logits_ref, route_ref,
                 flat_scr, w1b, w2b, w3b, w4b, w5b, w6b, w7b, w8b, w9b,
                 errc_scr, lcur_scr, rmin_scr, wsem, fsem):
    e = pl.program_id(0)
    b = pl.program_id(1)
    hbm = [We1_hbm, We2_hbm, We3_hbm, Wd1_hbm, Wd2_hbm, Wd3_hbm,
           Wx1_hbm, Wx2_hbm, Wx3_hbm]
    buf = [w1b, w2b, w3b, w4b, w5b, w6b, w7b, w8b, w9b]

    def wcopies(i, slot, ei):
        if i in (0, 5):
            d0 = _W_SHAPES[i][0] // 2
            return [
                pltpu.make_async_copy(hbm[i].at[ei, pl.ds(0, d0), :],
                                      buf[i].at[slot, pl.ds(0, d0), :],
                                      wsem.at[i]),
                pltpu.make_async_copy(hbm[i].at[ei, pl.ds(d0, d0), :],
                                      buf[i].at[slot, pl.ds(d0, d0), :],
                                      wsem.at[9 + (0 if i == 0 else 1)]),
            ]
        return [pltpu.make_async_copy(hbm[i].at[ei], buf[i].at[slot], wsem.at[i])]

    def fcopy(t):
        return pltpu.make_async_copy(
            flat_hbm.at[pl.ds(t * _BT, _BT), :],
            flat_scr.at[pl.ds(t * _BT, _BT), :], fsem.at[t])

    @pl.when((e == 0) & (b == 0))
    def _():
        for t in range(_NB):
            fcopy(t).start()
        for i in range(9):
            for c in wcopies(i, 0, 0):
                c.start()

    @pl.when(e == 0)
    def _():
        fcopy(b).wait()

    @pl.when(b == 0)
    def _():
        for i in range(9):
            for c in wcopies(i, e % 2, e):
                c.wait()

    @pl.when((b == 0) & (e < _E - 1))
    def _():
        for i in range(9):
            for c in wcopies(i, (e + 1) % 2, e + 1):
                c.start()

    s = e % 2
    rows = pl.ds(b * _BT, _BT)
    flat = flat_scr[rows, :]
    f32 = jnp.float32
    h = jnp.maximum(jnp.dot(flat, w1b[s], preferred_element_type=f32) + be1_ref[0], 0.0)
    h = jnp.maximum(jnp.dot(h, w2b[s], preferred_element_type=f32) + be2_ref[0], 0.0)
    lat = jnp.dot(h, w3b[s], preferred_element_type=f32) + be3_ref[0]
    d = jnp.maximum(jnp.dot(lat, w4b[s], preferred_element_type=f32) + bd1_ref[0], 0.0)
    d = jnp.maximum(jnp.dot(d, w5b[s], preferred_element_type=f32) + bd2_ref[0], 0.0)
    recon = jnp.dot(d, w6b[s], preferred_element_type=f32) + bd3_ref[0]
    recon_ref[0] = recon

    err = jnp.mean(jnp.abs(recon - flat), axis=1, keepdims=True)
    lane8 = jax.lax.broadcasted_iota(jnp.int32, (_BT, _E), 1)
    errc_scr[rows, :] = jnp.where(lane8 == e, err, errc_scr[rows, :])

    e1 = jnp.maximum(jnp.dot(lat, w7b[s], preferred_element_type=f32) + bx1_ref[0], 0.0)
    e1 = jnp.maximum(jnp.dot(e1, w8b[s], preferred_element_type=f32) + bx2_ref[0], 0.0)
    eo = jnp.dot(e1, w9b[s], preferred_element_type=f32) + bx3_ref[0]

    better = (err < rmin_scr[rows, :]) | (e == 0)
    rmin_scr[rows, :] = jnp.where(better, err, rmin_scr[rows, :])
    lcur_scr[rows, :] = jnp.where(better, eo, lcur_scr[rows, :])

    @pl.when((e == _E - 1) & (b == _NB - 1))
    def _():
        errc = errc_scr[...]
        min_v = errc[:, 0:1]
        min_i = jnp.zeros((_B, 1), jnp.int32)
        for k in range(1, _E):
            v = errc[:, k:k + 1]
            lt = v < min_v
            min_v = jnp.where(lt, v, min_v)
            min_i = jnp.where(lt, k, min_i)
        z = jnp.exp((min_v - errc) / _TEMP)
        rel = z / jnp.sum(z, axis=1, keepdims=True)
        laneB = jax.lax.broadcasted_iota(jnp.int32, (_B, 128), 1)
        mask_f = (laneB - _E == min_i).astype(f32)
        packed = jnp.concatenate([
            rel,
            jnp.zeros((_B, 120), f32)], axis=1)
        packed = jnp.where((laneB >= _E) & (laneB < 2 * _E), mask_f, packed)
        packed = jnp.where(laneB == 16, min_i.astype(f32), packed)
        packed = jnp.where(laneB == 17, min_v, packed)
        route_ref[...] = packed
        logits_ref[...] = lcur_scr[...]


def kernel(x, We1, be1, We2, be2, We3, be3, Wd1, bd1, Wd2, bd2, Wd3, bd3,
           Wx1, bx1, Wx2, bx2, Wx3, bx3):
    flat = x.reshape(_B, _D)
    b3 = lambda b: b.reshape(_E, 1, -1)

    anyspec = pl.BlockSpec(memory_space=pl.ANY)
    bspec = lambda n: pl.BlockSpec((1, 1, n), lambda e, b: (e, 0, 0))
    full = lambda *shape: pl.BlockSpec(shape, lambda e, b: (0,) * len(shape))

    recon, logits, route = pl.pallas_call(
        _expert_body,
        grid=(_E, _NB),
        in_specs=[anyspec] * 10 + [
            bspec(_HID), bspec(_HID), bspec(_LAT),
            bspec(_HID), bspec(_HID), bspec(_D),
            bspec(_HID), bspec(_HID), bspec(_CLS),
        ],
        out_specs=[
            pl.BlockSpec((1, _BT, _D), lambda e, b: (e, b, 0)),
            full(_B, _CLS), full(_B, 128),
        ],
        out_shape=[
            jax.ShapeDtypeStruct((_E, _B, _D), jnp.float32),
            jax.ShapeDtypeStruct((_B, _CLS), jnp.float32),
            jax.ShapeDtypeStruct((_B, 128), jnp.float32),
        ],
        scratch_shapes=[
            pltpu.VMEM((_B, _D), jnp.float32),
        ] + [pltpu.VMEM((2,) + s, jnp.float32) for s in _W_SHAPES] + [
            pltpu.VMEM((_B, _E), jnp.float32),
            pltpu.VMEM((_B, _CLS), jnp.float32),
            pltpu.VMEM((_B, 1), jnp.float32),
            pltpu.SemaphoreType.DMA((11,)),
            pltpu.SemaphoreType.DMA((_NB,)),
        ],
    )(flat, We1, We2, We3, Wd1, Wd2, Wd3, Wx1, Wx2, Wx3,
      b3(be1), b3(be2), b3(be3), b3(bd1), b3(bd2), b3(bd3),
      b3(bx1), b3(bx2), b3(bx3))

    reconstructions = recon.reshape(_E, _B, _C, _H, _W)
    rel = route[:, 0:_E].T
    mask = (route[:, _E:2 * _E].T != 0.0)
    indices = route[:, 16].astype(jnp.int32)
    min_err = route[:, 17]
    return (logits, reconstructions, indices, min_err, rel, mask)


# speedup vs baseline: 1.0088x; 1.0088x over previous
import jax
import jax.numpy as jnp
from jax.experimental import pallas as pl
from jax.experimental.pallas import tpu as pltpu

_E = 8
_B = 1024
_C, _H, _W = 3, 32, 32
_D = _C * _H * _W
_HID = 512
_LAT = 128
_CLS = 100
_TEMP = 2.0
_BT = 256
_NB = _B // _BT

_W_SHAPES = [(_D, _HID), (_HID, _HID), (_HID, _LAT),
             (_LAT, _HID), (_HID, _HID), (_HID, _D),
             (_LAT, _HID), (_HID, _HID), (_HID, _CLS)]


def _expert_body(flat_hbm, We1_hbm, We2_hbm, We3_hbm, Wd1_hbm, Wd2_hbm, Wd3_hbm,
                 Wx1_hbm, Wx2_hbm, Wx3_hbm,
                 be1_ref, be2_ref, be3_ref, bd1_ref, bd2_ref, bd3_ref,
                 bx1_ref, bx2_ref, bx3_ref,
                 recon_ref, err_ref, eo_ref,
                 flat_scr, w1b, w2b, w3b, w4b, w5b, w6b, w7b, w8b, w9b,
                 wsem, fsem):
    e = pl.program_id(0)
    b = pl.program_id(1)
    hbm = [We1_hbm, We2_hbm, We3_hbm, Wd1_hbm, Wd2_hbm, Wd3_hbm,
           Wx1_hbm, Wx2_hbm, Wx3_hbm]
    buf = [w1b, w2b, w3b, w4b, w5b, w6b, w7b, w8b, w9b]

    def wcopy(i, slot, ei):
        return pltpu.make_async_copy(hbm[i].at[ei], buf[i].at[slot], wsem.at[i])

    def fcopy(t):
        return pltpu.make_async_copy(
            flat_hbm.at[pl.ds(t * _BT, _BT), :],
            flat_scr.at[pl.ds(t * _BT, _BT), :], fsem.at[t])

    @pl.when((e == 0) & (b == 0))
    def _():
        for t in range(_NB):
            fcopy(t).start()
        for i in range(9):
            wcopy(i, 0, 0).start()

    @pl.when(e == 0)
    def _():
        fcopy(b).wait()

    @pl.when(b == 0)
    def _():
        for i in range(9):
            wcopy(i, e % 2, e).wait()

    @pl.when((b == 0) & (e < _E - 1))
    def _():
        for i in range(9):
            wcopy(i, (e + 1) % 2, e + 1).start()

    s = e % 2
    flat = flat_scr[pl.ds(b * _BT, _BT), :]
    f32 = jnp.float32
    h = jnp.maximum(jnp.dot(flat, w1b[s], preferred_element_type=f32) + be1_ref[0], 0.0)
    h = jnp.maximum(jnp.dot(h, w2b[s], preferred_element_type=f32) + be2_ref[0], 0.0)
    lat = jnp.dot(h, w3b[s], preferred_element_type=f32) + be3_ref[0]
    d = jnp.maximum(jnp.dot(lat, w4b[s], preferred_element_type=f32) + bd1_ref[0], 0.0)
    d = jnp.maximum(jnp.dot(d, w5b[s], preferred_element_type=f32) + bd2_ref[0], 0.0)
    recon = jnp.dot(d, w6b[s], preferred_element_type=f32) + bd3_ref[0]
    recon_ref[0] = recon
    err_ref[0, 0, pl.ds(b * _BT, _BT)] = jnp.mean(jnp.abs(recon - flat), axis=1)
    e1 = jnp.maximum(jnp.dot(lat, w7b[s], preferred_element_type=f32) + bx1_ref[0], 0.0)
    e1 = jnp.maximum(jnp.dot(e1, w8b[s], preferred_element_type=f32) + bx2_ref[0], 0.0)
    eo_ref[0] = jnp.dot(e1, w9b[s], preferred_element_type=f32) + bx3_ref[0]


def _route_body(err_ref, errc_ref, eo_ref, logits_ref, rel_ref, idx_ref, mine_ref, mask_ref):
    errs = err_ref[:, 0, :]
    min_v = errs[0:1, :]
    min_i = jnp.zeros((1, _B), jnp.int32)
    for e in range(1, _E):
        v = errs[e:e + 1, :]
        lt = v < min_v
        min_v = jnp.where(lt, v, min_v)
        min_i = jnp.where(lt, e, min_i)
    z = jnp.exp((min_v - errs) / _TEMP)
    rel_ref[...] = z / jnp.sum(z, axis=0, keepdims=True)
    eids = jax.lax.broadcasted_iota(jnp.int32, (_E, _B), 0)
    mask_ref[...] = (eids == min_i).astype(jnp.int32)
    idx_ref[...] = min_i
    mine_ref[...] = min_v
    errc = errc_ref[...]
    min_vc = errc[:, 0:1]
    min_ic = jnp.zeros((_B, 1), jnp.int32)
    for e in range(1, _E):
        v = errc[:, e:e + 1]
        lt = v < min_vc
        min_vc = jnp.where(lt, v, min_vc)
        min_ic = jnp.where(lt, e, min_ic)
    acc = jnp.zeros((_B, _CLS), jnp.float32)
    for e in range(_E):
        acc = acc + eo_ref[e] * (min_ic == e).astype(jnp.float32)
    logits_ref[...] = acc


def kernel(x, We1, be1, We2, be2, We3, be3, Wd1, bd1, Wd2, bd2, Wd3, bd3,
           Wx1, bx1, Wx2, bx2, Wx3, bx3):
    flat = x.reshape(_B, _D)
    b3 = lambda b: b.reshape(_E, 1, -1)

    anyspec = pl.BlockSpec(memory_space=pl.ANY)
    bspec = lambda n: pl.BlockSpec((1, 1, n), lambda e, b: (e, 0, 0))

    recon, errs, eo = pl.pallas_call(
        _expert_body,
        grid=(_E, _NB),
        in_specs=[anyspec] * 10 + [
            bspec(_HID), bspec(_HID), bspec(_LAT),
            bspec(_HID), bspec(_HID), bspec(_D),
            bspec(_HID), bspec(_HID), bspec(_CLS),
        ],
        out_specs=[
            pl.BlockSpec((1, _BT, _D), lambda e, b: (e, b, 0)),
            pl.BlockSpec((1, 1, _B), lambda e, b: (e, 0, 0)),
            pl.BlockSpec((1, _BT, _CLS), lambda e, b: (e, b, 0)),
        ],
        out_shape=[
            jax.ShapeDtypeStruct((_E, _B, _D), jnp.float32),
            jax.ShapeDtypeStruct((_E, 1, _B), jnp.float32),
            jax.ShapeDtypeStruct((_E, _B, _CLS), jnp.float32),
        ],
        scratch_shapes=[
            pltpu.VMEM((_B, _D), jnp.float32),
        ] + [pltpu.VMEM((2,) + s, jnp.float32) for s in _W_SHAPES] + [
            pltpu.SemaphoreType.DMA((9,)),
            pltpu.SemaphoreType.DMA((_NB,)),
        ],
    )(flat, We1, We2, We3, Wd1, Wd2, Wd3, Wx1, Wx2, Wx3,
      b3(be1), b3(be2), b3(be3), b3(bd1), b3(bd2), b3(bd3),
      b3(bx1), b3(bx2), b3(bx3))

    errs_col = jnp.swapaxes(errs.reshape(_E, _B), 0, 1)
    logits, rel, idx, mine, mask_i = pl.pallas_call(
        _route_body,
        out_shape=[
            jax.ShapeDtypeStruct((_B, _CLS), jnp.float32),
            jax.ShapeDtypeStruct((_E, _B), jnp.float32),
            jax.ShapeDtypeStruct((1, _B), jnp.int32),
            jax.ShapeDtypeStruct((1, _B), jnp.float32),
            jax.ShapeDtypeStruct((_E, _B), jnp.int32),
        ],
    )(errs, errs_col, eo)

    reconstructions = recon.reshape(_E, _B, _C, _H, _W)
    return (logits, reconstructions, idx.reshape(_B), mine.reshape(_B),
            rel, mask_i.astype(jnp.bool_))
